# baseline (speedup 1.0000x reference)
import functools

import jax
import jax.numpy as jnp
from jax import lax
from jax.experimental import pallas as pl
from jax.experimental.pallas import tpu as pltpu
from jax.experimental.pallas import tpu_sc as plsc

_MAX_POS = 8192
_EMBEDDING_DIM = 16
_NUM_CORES = 2
_NUM_SUBCORES = 16
_NUM_WORKERS = _NUM_CORES * _NUM_SUBCORES
_ROWS_PER_WORKER = _MAX_POS // _NUM_WORKERS


@functools.partial(
    pl.kernel,
    out_type=jax.ShapeDtypeStruct((_MAX_POS, _EMBEDDING_DIM), jnp.float32),
    mesh=plsc.VectorSubcoreMesh(core_axis_name="c", subcore_axis_name="s"),
    scratch_types=[pltpu.VMEM((_ROWS_PER_WORKER, _EMBEDDING_DIM), jnp.float32)],
)
def _sc_row_copy(w_hbm, out_hbm, rows_v):
    wid = lax.axis_index("s") * _NUM_CORES + lax.axis_index("c")
    base = wid * _ROWS_PER_WORKER
    pltpu.sync_copy(w_hbm.at[pl.ds(base, _ROWS_PER_WORKER)], rows_v)
    pltpu.sync_copy(rows_v, out_hbm.at[pl.ds(base, _ROWS_PER_WORKER)])


def kernel(input, weights):
    del input
    return _sc_row_copy(weights)
